# baseline (speedup 1.0000x reference)
import functools

import jax
import jax.numpy as jnp
from jax import lax
from jax.experimental import pallas as pl
from jax.experimental.pallas import tpu as pltpu
from jax.experimental.pallas import tpu_sc as plsc

K = 5
ROWS = 128
VOCAB = 100000
CHUNK = 10000
NCHUNK = VOCAB // CHUNK


def _sc_count_kernel():
    info = plsc.get_sparse_core_info()
    nc, ns, nl = info.num_cores, info.num_subcores, info.num_lanes
    nw = nc * ns
    rows_per_w = ROWS // nw

    mesh = plsc.VectorSubcoreMesh(core_axis_name="c", subcore_axis_name="s")

    @functools.partial(
        pl.kernel,
        mesh=mesh,
        compiler_params=pltpu.CompilerParams(needs_layout_passes=False),
        out_type=jax.ShapeDtypeStruct((nw, 128), jnp.float32),
        scratch_types=[
            pltpu.VMEM((ROWS,), jnp.int32),
            pltpu.VMEM((16,), jnp.int32),
            pltpu.VMEM((16,), jnp.float32),
            pltpu.VMEM((CHUNK,), jnp.float32),
            pltpu.VMEM((128,), jnp.float32),
            pltpu.SemaphoreType.DMA,
        ],
    )
    def sc_k(y_hbm, lbl_hbm, out_hbm, lbl_v, idx_v, vv, buf, ostage, sem):
        wid = lax.axis_index("s") * nc + lax.axis_index("c")
        base_row = wid * rows_per_w
        lanes = lax.iota(jnp.int32, 16)

        pltpu.sync_copy(lbl_hbm, lbl_v)

        rows16 = base_row + (lanes & (rows_per_w - 1))
        lbl16 = plsc.load_gather(lbl_v, [rows16])
        idx_v[...] = rows16 * VOCAB + lbl16
        pltpu.async_copy(y_hbm.at[idx_v], vv, sem).wait()

        cnt = jnp.float32(0.0)
        for r in range(rows_per_w):
            rsel = jnp.full((16,), r, jnp.int32)
            v_b = plsc.load_gather(vv, [rsel])
            lab_b = plsc.load_gather(lbl_v, [jnp.full((16,), r, jnp.int32) + base_row])
            flat_base = (base_row + r) * VOCAB
            acc = jnp.zeros((16,), jnp.int32)
            for c in range(NCHUNK):
                pltpu.sync_copy(y_hbm.at[pl.ds(flat_base + c * CHUNK, CHUNK)], buf)
                col0 = c * CHUNK + lanes

                def body(i, carry):
                    a, col = carry
                    x = buf[pl.ds(i * 16, 16)]
                    m = (x > v_b) | ((x == v_b) & (col < lab_b))
                    return a + m.astype(jnp.int32), col + 16

                acc, _ = lax.fori_loop(0, CHUNK // 16, body, (acc, col0))
            rank = jnp.sum(acc)
            cnt = cnt + jnp.where(rank < K, jnp.float32(1.0), jnp.float32(0.0))

        splat = jnp.full((16,), cnt, jnp.float32)
        zeros = jnp.zeros((16,), jnp.float32)
        for j in range(8):
            ostage[pl.ds(j * 16, 16)] = splat if j == 0 else zeros
        pltpu.sync_copy(ostage, out_hbm.at[wid])

    return sc_k, nw


def _tc_reduce(partials):
    def body(x_ref, o_ref):
        o_ref[...] = jnp.sum(x_ref[...], axis=(0, 1), keepdims=True) * jnp.float32(
            1.0 / (16 * ROWS)
        )

    return pl.pallas_call(
        body,
        out_shape=jax.ShapeDtypeStruct((1, 1), jnp.float32),
    )(partials)


def kernel(y_probs, y_true_label):
    y_flat = jnp.reshape(y_probs, (-1,))
    labels = y_true_label.astype(jnp.int32)
    sc_k, nw = _sc_count_kernel()
    partials = sc_k(y_flat, labels)
    return _tc_reduce(partials)[0, 0]


# speedup vs baseline: 1.6446x; 1.6446x over previous
import functools

import jax
import jax.numpy as jnp
from jax import lax
from jax.experimental import pallas as pl
from jax.experimental.pallas import tpu as pltpu
from jax.experimental.pallas import tpu_sc as plsc

K = 5
ROWS = 128
VOCAB = 100000
HALF = VOCAB // 2
HVEC = HALF // 16
UNROLL = 8


def _sc_count_kernel():
    info = plsc.get_sparse_core_info()
    nc, ns = info.num_cores, info.num_subcores
    nw = nc * ns
    rows_per_w = ROWS // nw
    nhalf = rows_per_w * 2

    mesh = plsc.VectorSubcoreMesh(core_axis_name="c", subcore_axis_name="s")

    @functools.partial(
        pl.kernel,
        mesh=mesh,
        compiler_params=pltpu.CompilerParams(needs_layout_passes=False),
        out_type=jax.ShapeDtypeStruct((nw, 128), jnp.float32),
        scratch_types=[
            pltpu.VMEM((ROWS,), jnp.int32),
            pltpu.VMEM((16,), jnp.int32),
            pltpu.VMEM((16,), jnp.float32),
            pltpu.VMEM((HALF,), jnp.float32),
            pltpu.VMEM((HALF,), jnp.float32),
            pltpu.VMEM((128,), jnp.float32),
            pltpu.SemaphoreType.DMA,
            pltpu.SemaphoreType.DMA,
        ],
    )
    def sc_k(y_hbm, lbl_hbm, out_hbm, lbl_v, idx_v, vv, buf_a, buf_b, ostage,
             sem_a, sem_b):
        wid = lax.axis_index("s") * nc + lax.axis_index("c")
        base_row = wid * rows_per_w
        lanes = lax.iota(jnp.int32, 16)
        bufs = (buf_a, buf_b)
        sems = (sem_a, sem_b)

        pltpu.sync_copy(lbl_hbm, lbl_v)

        rows16 = base_row + (lanes & (rows_per_w - 1))
        lbl16 = plsc.load_gather(lbl_v, [rows16])
        idx_v[...] = rows16 * VOCAB + lbl16
        pltpu.async_copy(y_hbm.at[idx_v], vv, sem_a).wait()

        tile_flat = base_row * VOCAB

        def start(g):
            return pltpu.async_copy(
                y_hbm.at[pl.ds(tile_flat + g * HALF, HALF)],
                bufs[g & 1], sems[g & 1])

        def count_span(buf, lo, hi, cmp_fn, acc):
            n = hi - lo
            n_main = (n >> 3) << 3

            def body_u(i, a):
                base = lo + i * UNROLL
                for u in range(UNROLL):
                    x = buf[pl.ds((base + u) * 16, 16)]
                    a = a + plsc.all_reduce_population_count(cmp_fn(x))
                return a

            acc = lax.fori_loop(0, n >> 3, body_u, acc)

            def body_1(i, a):
                x = buf[pl.ds(i * 16, 16)]
                return a + plsc.all_reduce_population_count(cmp_fn(x))

            return lax.fori_loop(lo + n_main, hi, body_1, acc)

        handles = [None, None]
        handles[0] = start(0)
        cnt = jnp.float32(0.0)
        acc = jnp.zeros((16,), jnp.int32)
        for g in range(nhalf):
            r = g // 2
            base_v = (g & 1) * HVEC
            if g + 1 < nhalf:
                handles[(g + 1) & 1] = start(g + 1)
            handles[g & 1].wait()
            buf = bufs[g & 1]

            rsel = jnp.full((16,), r, jnp.int32)
            v_b = plsc.load_gather(vv, [rsel])
            lab_b = plsc.load_gather(lbl_v, [rsel + base_row])
            l_s = jnp.max(lab_b)
            lv = (l_s >> 4) - base_v

            a_end = jnp.clip(lv, 0, HVEC)
            has_mid = (lv >= 0) & (lv < HVEC)
            b_start = jnp.where(has_mid, a_end + 1, a_end)

            if g & 1 == 0:
                acc = jnp.zeros((16,), jnp.int32)
            acc = count_span(buf, jnp.int32(0), a_end, lambda x: x >= v_b, acc)

            mid_v = jnp.minimum(a_end, HVEC - 1)
            x_mid = buf[pl.ds(mid_v * 16, 16)]
            col_mid = (base_v + mid_v) * 16 + lanes
            m_mid = (x_mid > v_b) | ((x_mid == v_b) & (col_mid < lab_b))
            mid_cnt = plsc.all_reduce_population_count(m_mid)
            acc = acc + mid_cnt * jnp.where(has_mid, 1, 0).astype(jnp.int32)

            acc = count_span(buf, b_start, jnp.int32(HVEC), lambda x: x > v_b,
                             acc)

            if g & 1 == 1:
                rank16 = jnp.sum(acc)
                cnt = cnt + jnp.where(rank16 < 16 * K, jnp.float32(1.0),
                                      jnp.float32(0.0))

        splat = jnp.full((16,), cnt, jnp.float32)
        zeros = jnp.zeros((16,), jnp.float32)
        for j in range(8):
            ostage[pl.ds(j * 16, 16)] = splat if j == 0 else zeros
        pltpu.sync_copy(ostage, out_hbm.at[wid])

    return sc_k, nw


def _tc_reduce(partials):
    def body(x_ref, o_ref):
        o_ref[...] = jnp.sum(x_ref[...], axis=(0, 1), keepdims=True) * jnp.float32(
            1.0 / (16 * ROWS)
        )

    return pl.pallas_call(
        body,
        out_shape=jax.ShapeDtypeStruct((1, 1), jnp.float32),
    )(partials)


def kernel(y_probs, y_true_label):
    y_flat = jnp.reshape(y_probs, (-1,))
    labels = y_true_label.astype(jnp.int32)
    sc_k, nw = _sc_count_kernel()
    partials = sc_k(y_flat, labels)
    return _tc_reduce(partials)[0, 0]
